# baseline (speedup 1.0000x reference)
import math

import jax
import jax.numpy as jnp
from jax.experimental import pallas as pl
from jax.experimental.pallas import tpu as pltpu

_N = 8192
_BR = 256


def _body(pr, tr, pc, tc, out):
    d = pr[...] - pc[...]
    lab_d = jnp.where(tr[...] > tc[...], d, 0.0)
    bce = jnp.maximum(d, 0.0) + jnp.log1p(jnp.exp(-jnp.abs(d))) - lab_d
    out[0, 0, 0] = jnp.sum(bce)


def _partial_sums(p_row, t_row, p_col, t_col):
    grid = _N // _BR
    return pl.pallas_call(
        _body,
        grid=(grid,),
        in_specs=[
            pl.BlockSpec((_BR, 1), lambda i: (i, 0)),
            pl.BlockSpec((_BR, 1), lambda i: (i, 0)),
            pl.BlockSpec((1, _N), lambda i: (0, 0)),
            pl.BlockSpec((1, _N), lambda i: (0, 0)),
        ],
        out_specs=pl.BlockSpec((1, 1, 1), lambda i: (i, 0, 0),
                               memory_space=pltpu.SMEM),
        out_shape=jax.ShapeDtypeStruct((grid, 1, 1), jnp.float32),
        compiler_params=pltpu.CompilerParams(
            dimension_semantics=("parallel",),
        ),
    )(p_row, t_row, p_col, t_col)


def kernel(preds, targets):
    n = preds.shape[0]
    p_row = preds.reshape(n, 1)
    t_row = targets.reshape(n, 1)
    p_col = preds.reshape(1, n)
    t_col = targets.reshape(1, n)
    partials = _partial_sums(p_row, t_row, p_col, t_col)
    total = jnp.sum(partials) - n * math.log(2.0)
    return total / (n * (n - 1))


# speedup vs baseline: 1.2126x; 1.2126x over previous
import jax
import jax.numpy as jnp
import numpy as np
from jax.experimental import pallas as pl
from jax.experimental.pallas import tpu as pltpu

_N = 8192
_B = 512
_T = _N // _B
_NBLK = _T * (_T + 1) // 2

_RMAP, _CMAP = (np.array(x, dtype=np.int32) for x in zip(
    *[(r, c) for r in range(_T) for c in range(r, _T)]))


def _body(rmap, cmap, pr, tr, pc, tc, out):
    i = pl.program_id(0)
    r = rmap[i]
    c = cmap[i]
    d = pr[...] - pc[...]
    td = tr[...] - tc[...]
    a = jnp.abs(d)
    cm = (a + 2.0 * jnp.log1p(jnp.exp(-a))
          - jnp.where(td > 0, d, 0.0) + jnp.where(td < 0, d, 0.0))

    @pl.when(r == c)
    def _diag():
        ii = jax.lax.broadcasted_iota(jnp.int32, (_B, _B), 0)
        jj = jax.lax.broadcasted_iota(jnp.int32, (_B, _B), 1)
        out[0, 0, 0] = jnp.sum(jnp.where(jj > ii, cm, 0.0))

    @pl.when(r != c)
    def _full():
        out[0, 0, 0] = jnp.sum(cm)


def _partial_sums(p_row, t_row, p_col, t_col):
    return pl.pallas_call(
        _body,
        grid_spec=pltpu.PrefetchScalarGridSpec(
            num_scalar_prefetch=2,
            grid=(_NBLK,),
            in_specs=[
                pl.BlockSpec((_B, 1), lambda i, rm, cm: (rm[i], 0)),
                pl.BlockSpec((_B, 1), lambda i, rm, cm: (rm[i], 0)),
                pl.BlockSpec((1, _B), lambda i, rm, cm: (0, cm[i])),
                pl.BlockSpec((1, _B), lambda i, rm, cm: (0, cm[i])),
            ],
            out_specs=pl.BlockSpec((1, 1, 1), lambda i, rm, cm: (i, 0, 0),
                                   memory_space=pltpu.SMEM),
        ),
        out_shape=jax.ShapeDtypeStruct((_NBLK, 1, 1), jnp.float32),
        compiler_params=pltpu.CompilerParams(
            dimension_semantics=("parallel",),
        ),
    )(jnp.asarray(_RMAP), jnp.asarray(_CMAP), p_row, t_row, p_col, t_col)


def kernel(preds, targets):
    n = preds.shape[0]
    p_row = preds.reshape(n, 1)
    t_row = targets.reshape(n, 1)
    p_col = preds.reshape(1, n)
    t_col = targets.reshape(1, n)
    partials = _partial_sums(p_row, t_row, p_col, t_col)
    return jnp.sum(partials) / (n * (n - 1))


# speedup vs baseline: 1.8716x; 1.5434x over previous
import jax
import jax.numpy as jnp
import numpy as np
from jax.experimental import pallas as pl
from jax.experimental.pallas import tpu as pltpu

_N = 8192
_B = 512
_T = _N // _B
_NBLK = _T * (_T + 1) // 2

_RMAP, _CMAP = (np.array(x, dtype=np.int32) for x in zip(
    *[(r, c) for r in range(_T) for c in range(r, _T)]))

_NEG_LOG2E = -1.4426950408889634
_TWO_LN2 = 1.3862943611198906


def _body(rmap, cmap, pr, tr, pc, tc, out, acc):
    i = pl.program_id(0)
    r = rmap[i]
    c = cmap[i]

    @pl.when(i == 0)
    def _init():
        acc[...] = jnp.zeros_like(acc)

    d = pr[...] - pc[...]
    a = jnp.abs(d)
    g = jnp.log2(1.0 + jnp.exp2(a * _NEG_LOG2E))
    cm = (a + _TWO_LN2 * g
          - jnp.where(tr[...] > tc[...], d, 0.0)
          + jnp.where(tr[...] < tc[...], d, 0.0))

    @pl.when(r == c)
    def _diag():
        ii = jax.lax.broadcasted_iota(jnp.int32, (_B, _B), 0)
        jj = jax.lax.broadcasted_iota(jnp.int32, (_B, _B), 1)
        acc[...] += jnp.where(jj > ii, cm, 0.0)

    @pl.when(r != c)
    def _full():
        acc[...] += cm

    @pl.when(i == _NBLK - 1)
    def _fold():
        s = acc[...].reshape(_B // 8, 8, _B // 128, 128)
        out[...] = jnp.sum(s, axis=(0, 2))


def _acc_sums(p_row, t_row, p_col, t_col):
    return pl.pallas_call(
        _body,
        grid_spec=pltpu.PrefetchScalarGridSpec(
            num_scalar_prefetch=2,
            grid=(_NBLK,),
            in_specs=[
                pl.BlockSpec((_B, 1), lambda i, rm, cm: (rm[i], 0)),
                pl.BlockSpec((_B, 1), lambda i, rm, cm: (rm[i], 0)),
                pl.BlockSpec((1, _B), lambda i, rm, cm: (0, cm[i])),
                pl.BlockSpec((1, _B), lambda i, rm, cm: (0, cm[i])),
            ],
            out_specs=pl.BlockSpec((8, 128), lambda i, rm, cm: (0, 0)),
            scratch_shapes=[pltpu.VMEM((_B, _B), jnp.float32)],
        ),
        out_shape=jax.ShapeDtypeStruct((8, 128), jnp.float32),
        compiler_params=pltpu.CompilerParams(
            dimension_semantics=("arbitrary",),
        ),
    )(jnp.asarray(_RMAP), jnp.asarray(_CMAP), p_row, t_row, p_col, t_col)


def kernel(preds, targets):
    n = preds.shape[0]
    p_row = preds.reshape(n, 1)
    t_row = targets.reshape(n, 1)
    p_col = preds.reshape(1, n)
    t_col = targets.reshape(1, n)
    acc = _acc_sums(p_row, t_row, p_col, t_col)
    return jnp.sum(acc) / (n * (n - 1))


# speedup vs baseline: 2.0292x; 1.0842x over previous
import jax
import jax.numpy as jnp
import numpy as np
from jax.experimental import pallas as pl
from jax.experimental.pallas import tpu as pltpu

_N = 8192
_B = 1024
_T = _N // _B
_NBLK = _T * (_T + 1) // 2

_RMAP, _CMAP = (np.array(x, dtype=np.int32) for x in zip(
    *[(r, c) for r in range(_T) for c in range(r, _T)]))

_NEG_LOG2E = -1.4426950408889634
_TWO_LN2 = 1.3862943611198906


def _body(rmap, cmap, pr, tr, pc, tc, out, acc):
    i = pl.program_id(0)
    r = rmap[i]
    c = cmap[i]

    @pl.when(i == 0)
    def _init():
        acc[...] = jnp.zeros_like(acc)

    d = pr[...] - pc[...]
    a = jnp.abs(d)
    g = jnp.log2(1.0 + jnp.exp2(a * _NEG_LOG2E))
    cm = (a + _TWO_LN2 * g
          - jnp.where(tr[...] > tc[...], d, 0.0)
          + jnp.where(tr[...] < tc[...], d, 0.0))

    @pl.when(r == c)
    def _diag():
        ii = jax.lax.broadcasted_iota(jnp.int32, (_B, _B), 0)
        jj = jax.lax.broadcasted_iota(jnp.int32, (_B, _B), 1)
        acc[...] += jnp.where(jj > ii, cm, 0.0)

    @pl.when(r != c)
    def _full():
        acc[...] += cm

    @pl.when(i == _NBLK - 1)
    def _fold():
        s = acc[...].reshape(_B // 8, 8, _B // 128, 128)
        out[...] = jnp.sum(s, axis=(0, 2))


def _acc_sums(p_row, t_row, p_col, t_col):
    return pl.pallas_call(
        _body,
        grid_spec=pltpu.PrefetchScalarGridSpec(
            num_scalar_prefetch=2,
            grid=(_NBLK,),
            in_specs=[
                pl.BlockSpec((_B, 1), lambda i, rm, cm: (rm[i], 0)),
                pl.BlockSpec((_B, 1), lambda i, rm, cm: (rm[i], 0)),
                pl.BlockSpec((1, _B), lambda i, rm, cm: (0, cm[i])),
                pl.BlockSpec((1, _B), lambda i, rm, cm: (0, cm[i])),
            ],
            out_specs=pl.BlockSpec((8, 128), lambda i, rm, cm: (0, 0)),
            scratch_shapes=[pltpu.VMEM((_B, _B), jnp.float32)],
        ),
        out_shape=jax.ShapeDtypeStruct((8, 128), jnp.float32),
        compiler_params=pltpu.CompilerParams(
            dimension_semantics=("arbitrary",),
        ),
    )(jnp.asarray(_RMAP), jnp.asarray(_CMAP), p_row, t_row, p_col, t_col)


def kernel(preds, targets):
    n = preds.shape[0]
    p_row = preds.reshape(n, 1)
    t_row = targets.reshape(n, 1)
    p_col = preds.reshape(1, n)
    t_col = targets.reshape(1, n)
    acc = _acc_sums(p_row, t_row, p_col, t_col)
    return jnp.sum(acc) / (n * (n - 1))


# speedup vs baseline: 2.4005x; 1.1830x over previous
import jax
import jax.numpy as jnp
import numpy as np
from jax.experimental import pallas as pl
from jax.experimental.pallas import tpu as pltpu

_N = 8192
_B = 1024
_T = _N // _B
_NBLK = _T * (_T + 1) // 2

_RMAP, _CMAP = (np.array(x, dtype=np.int32) for x in zip(
    *[(r, c) for r in range(_T) for c in range(r, _T)]))

_NEG_LOG2E = -1.4426950408889634
_TWO_LN2 = 1.3862943611198906


def _body(rmap, cmap, pr, tr, pc, tc, out, acc):
    i = pl.program_id(0)
    r = rmap[i]
    c = cmap[i]

    @pl.when(i == 0)
    def _init():
        acc[...] = jnp.zeros_like(acc)

    pc_v = pc[...]
    tc_v = tc[...]

    def strip(k, mask):
        pr_s = pr[8 * k:8 * k + 8, :]
        tr_s = tr[8 * k:8 * k + 8, :]
        d = pr_s - pc_v
        a = jnp.abs(d)
        g = jnp.log2(1.0 + jnp.exp2(a * _NEG_LOG2E))
        cm = (a + _TWO_LN2 * g
              - jnp.where(tr_s > tc_v, d, 0.0)
              + jnp.where(tr_s < tc_v, d, 0.0))
        if mask is not None:
            cm = jnp.where(mask > 8 * k, cm, 0.0)
        acc[8 * k:8 * k + 8, :] += cm

    @pl.when(r == c)
    def _diag():
        ii = jax.lax.broadcasted_iota(jnp.int32, (8, _B), 0)
        jj = jax.lax.broadcasted_iota(jnp.int32, (8, _B), 1)
        m0 = jj - ii
        for k in range(_B // 8):
            strip(k, m0)

    @pl.when(r != c)
    def _full():
        for k in range(_B // 8):
            strip(k, None)

    @pl.when(i == _NBLK - 1)
    def _fold():
        s = acc[...].reshape(_B // 8, 8, _B // 128, 128)
        out[...] = jnp.sum(s, axis=(0, 2))


def _acc_sums(p_row, t_row, p_col, t_col):
    return pl.pallas_call(
        _body,
        grid_spec=pltpu.PrefetchScalarGridSpec(
            num_scalar_prefetch=2,
            grid=(_NBLK,),
            in_specs=[
                pl.BlockSpec((_B, 1), lambda i, rm, cm: (rm[i], 0)),
                pl.BlockSpec((_B, 1), lambda i, rm, cm: (rm[i], 0)),
                pl.BlockSpec((1, _B), lambda i, rm, cm: (0, cm[i])),
                pl.BlockSpec((1, _B), lambda i, rm, cm: (0, cm[i])),
            ],
            out_specs=pl.BlockSpec((8, 128), lambda i, rm, cm: (0, 0)),
            scratch_shapes=[pltpu.VMEM((_B, _B), jnp.float32)],
        ),
        out_shape=jax.ShapeDtypeStruct((8, 128), jnp.float32),
        compiler_params=pltpu.CompilerParams(
            dimension_semantics=("arbitrary",),
        ),
    )(jnp.asarray(_RMAP), jnp.asarray(_CMAP), p_row, t_row, p_col, t_col)


def kernel(preds, targets):
    n = preds.shape[0]
    p_row = preds.reshape(n, 1)
    t_row = targets.reshape(n, 1)
    p_col = preds.reshape(1, n)
    t_col = targets.reshape(1, n)
    acc = _acc_sums(p_row, t_row, p_col, t_col)
    return jnp.sum(acc) / (n * (n - 1))


# speedup vs baseline: 2.4358x; 1.0147x over previous
import jax
import jax.numpy as jnp
import numpy as np
from jax.experimental import pallas as pl
from jax.experimental.pallas import tpu as pltpu

_N = 8192
_B = 1024
_T = _N // _B
_NBLK = _T * (_T + 1) // 2

_RMAP, _CMAP = (np.array(x, dtype=np.int32) for x in zip(
    *[(r, c) for r in range(_T) for c in range(r, _T)]))

_NEG_LOG2E = -1.4426950408889634
_TWO_LN2 = 1.3862943611198906
_LN2 = 0.6931471805599453


def _body(rmap, cmap, pr, tr, pc, tc, out, acc):
    i = pl.program_id(0)
    r = rmap[i]
    c = cmap[i]

    @pl.when(i == 0)
    def _init():
        acc[...] = jnp.zeros_like(acc)

    pc_v = pc[...]
    tc_v = tc[...]

    zeros = jnp.zeros((8, _B), jnp.float32)
    m_acc = [zeros, zeros]
    g_acc = [zeros, zeros]
    for k in range(_B // 8):
        pr_s = pr[8 * k:8 * k + 8, :]
        tr_s = tr[8 * k:8 * k + 8, :]
        d = pr_s - pc_v
        a = jnp.abs(d)
        g = jnp.log2(1.0 + jnp.exp2(a * _NEG_LOG2E))
        x = (a
             - jnp.where(tr_s > tc_v, d, 0.0)
             + jnp.where(tr_s < tc_v, d, 0.0))
        m_acc[k % 2] = m_acc[k % 2] + x
        g_acc[k % 2] = g_acc[k % 2] + g
    xm = m_acc[0] + m_acc[1]
    xg = g_acc[0] + g_acc[1]

    @pl.when(r != c)
    def _main():
        acc[0:8, :] += xm
        acc[8:16, :] += xg

    @pl.when(r == c)
    def _diag():
        acc[16:24, :] += xm
        acc[24:32, :] += xg

    @pl.when(i == _NBLK - 1)
    def _fold():
        def fold8(a8):
            tot = a8[:, 0:128]
            for l in range(1, _B // 128):
                tot = tot + a8[:, 128 * l:128 * (l + 1)]
            return tot

        lin = fold8(acc[0:8, :]) + 0.5 * fold8(acc[16:24, :])
        lg = fold8(acc[8:16, :]) + 0.5 * fold8(acc[24:32, :])
        out[...] = lin + _TWO_LN2 * lg


def _acc_sums(p_row, t_row, p_col, t_col):
    return pl.pallas_call(
        _body,
        grid_spec=pltpu.PrefetchScalarGridSpec(
            num_scalar_prefetch=2,
            grid=(_NBLK,),
            in_specs=[
                pl.BlockSpec((_B, 1), lambda i, rm, cm: (rm[i], 0)),
                pl.BlockSpec((_B, 1), lambda i, rm, cm: (rm[i], 0)),
                pl.BlockSpec((1, _B), lambda i, rm, cm: (0, cm[i])),
                pl.BlockSpec((1, _B), lambda i, rm, cm: (0, cm[i])),
            ],
            out_specs=pl.BlockSpec((8, 128), lambda i, rm, cm: (0, 0)),
            scratch_shapes=[pltpu.VMEM((32, _B), jnp.float32)],
        ),
        out_shape=jax.ShapeDtypeStruct((8, 128), jnp.float32),
        compiler_params=pltpu.CompilerParams(
            dimension_semantics=("arbitrary",),
        ),
    )(jnp.asarray(_RMAP), jnp.asarray(_CMAP), p_row, t_row, p_col, t_col)


def kernel(preds, targets):
    n = preds.shape[0]
    p_row = preds.reshape(n, 1)
    t_row = targets.reshape(n, 1)
    p_col = preds.reshape(1, n)
    t_col = targets.reshape(1, n)
    acc = _acc_sums(p_row, t_row, p_col, t_col)
    return (jnp.sum(acc) - n * _LN2) / (n * (n - 1))
